# baseline (speedup 1.0000x reference)
import jax
import jax.numpy as jnp
from jax.experimental import pallas as pl
from jax.experimental.pallas import tpu as pltpu

_MIN_NORM = 1e-15
_EPS = 1e-7

_N, _D = 10000, 128
_BM = 200
_NB = _N // _BM
_CB = 8


def _logmap0_kernel(x_ref, o_ref):
    p = x_ref[...]
    p0 = p[:, 0:1]
    y_sq = jnp.sum(p * p, axis=1, keepdims=True) - p0 * p0
    y_norm = jnp.sqrt(jnp.clip(y_sq, _MIN_NORM * _MIN_NORM, None))
    th = jnp.clip(p0, 1.0 + _EPS, None)
    ar = jnp.log(jnp.clip(th + jnp.sqrt(th * th - 1.0), _MIN_NORM, None))
    s = ar / y_norm
    col = jax.lax.broadcasted_iota(jnp.int32, p.shape, 1)
    o_ref[...] = jnp.where(col == 0, 0.0, p * s)


def _expmap0_proj(u):
    u0 = u[:, 0:1]
    x_sq = jnp.sum(u * u, axis=1, keepdims=True) - u0 * u0
    x_norm = jnp.sqrt(jnp.clip(x_sq, _MIN_NORM * _MIN_NORM, None))
    theta = jnp.clip(x_norm, -15.0, 15.0)
    e = jnp.exp(theta)
    sinh = 0.5 * (e - 1.0 / e)
    scale = sinh / x_norm
    y_sq_new = scale * scale * x_sq
    first = jnp.sqrt(jnp.clip(1.0 + y_sq_new, _EPS, None))
    col = jax.lax.broadcasted_iota(jnp.int32, u.shape, 1)
    return jnp.where(col == 0, first, u * scale)


def _p2_block(k, nb, cb):
    return jnp.where(k < 2 * cb, jnp.where(k % 2 == 0, cb + k // 2, k // 2), k)


def _gcn_kernel(adj_ref, xt_ref, h_ref, cache, s_ref, s16_ref):
    g = pl.program_id(0)
    nb, bm, cb = _NB, _BM, _CB

    @pl.when(g < nb)
    def _():
        i = g

        @pl.when(i < cb)
        def _():
            cache[i] = adj_ref[...].astype(jnp.bfloat16)

        acc = jnp.dot(adj_ref[...], xt_ref[...], preferred_element_type=jnp.float32)
        s_ref[pl.ds(i * bm, bm), :] = acc + xt_ref[pl.ds(i * bm, bm), :]

    @pl.when(g == nb - 1)
    def _():
        s16_ref[...] = s_ref[...].astype(jnp.bfloat16)

    @pl.when(g >= nb)
    def _():
        k = g - nb
        is_cached = jnp.logical_and(k < 2 * cb, k % 2 == 1)

        def residual(acc, i):
            u = acc + s_ref[pl.ds(i * bm, bm), :] - xt_ref[pl.ds(i * bm, bm), :]
            h_ref[...] = _expmap0_proj(u)

        @pl.when(is_cached)
        def _():
            j = k // 2
            acc = jnp.dot(cache[j], s16_ref[...], preferred_element_type=jnp.float32)
            residual(acc, j)

        @pl.when(jnp.logical_not(is_cached))
        def _():
            i = _p2_block(k, nb, cb)
            acc = jnp.dot(adj_ref[...], s_ref[...], preferred_element_type=jnp.float32)
            residual(acc, i)


def kernel(x, adj):
    n, d = x.shape
    nb, bm, cb = _NB, _BM, _CB
    bp = 1000
    xt = pl.pallas_call(
        _logmap0_kernel,
        grid=(n // bp,),
        in_specs=[pl.BlockSpec((bp, d), lambda i: (i, 0))],
        out_specs=pl.BlockSpec((bp, d), lambda i: (i, 0)),
        out_shape=jax.ShapeDtypeStruct((n, d), jnp.float32),
    )(x)

    def adj_index(g):
        k = g - nb
        return (jnp.where(g < nb, g, jnp.where(k < 2 * cb, cb + k // 2, k)), 0)

    def h_index(g):
        k = g - nb
        return (jnp.where(g < nb, cb, _p2_block(k, nb, cb)), 0)

    h = pl.pallas_call(
        _gcn_kernel,
        grid=(2 * nb,),
        in_specs=[
            pl.BlockSpec((bm, n), adj_index),
            pl.BlockSpec((n, d), lambda g: (0, 0)),
        ],
        out_specs=pl.BlockSpec((bm, d), h_index),
        out_shape=jax.ShapeDtypeStruct((n, d), jnp.float32),
        scratch_shapes=[
            pltpu.VMEM((_CB, _BM, _N), jnp.bfloat16),
            pltpu.VMEM((_N, _D), jnp.float32),
            pltpu.VMEM((_N, _D), jnp.bfloat16),
        ],
        compiler_params=pltpu.CompilerParams(
            dimension_semantics=("arbitrary",),
            vmem_limit_bytes=67000000,
        ),
    )(adj, xt)
    return h


# speedup vs baseline: 1.0395x; 1.0395x over previous
import jax
import jax.numpy as jnp
from jax.experimental import pallas as pl
from jax.experimental.pallas import tpu as pltpu

_MIN_NORM = 1e-15
_EPS = 1e-7

_N, _D = 10000, 128
_BM = 200
_NB = _N // _BM
_CB = 9


def _logmap0_kernel(x_ref, o_ref):
    p = x_ref[...]
    p0 = p[:, 0:1]
    y_sq = jnp.sum(p * p, axis=1, keepdims=True) - p0 * p0
    y_norm = jnp.sqrt(jnp.clip(y_sq, _MIN_NORM * _MIN_NORM, None))
    th = jnp.clip(p0, 1.0 + _EPS, None)
    ar = jnp.log(jnp.clip(th + jnp.sqrt(th * th - 1.0), _MIN_NORM, None))
    s = ar / y_norm
    col = jax.lax.broadcasted_iota(jnp.int32, p.shape, 1)
    o_ref[...] = jnp.where(col == 0, 0.0, p * s)


def _expmap0_proj(u):
    u0 = u[:, 0:1]
    x_sq = jnp.sum(u * u, axis=1, keepdims=True) - u0 * u0
    x_norm = jnp.sqrt(jnp.clip(x_sq, _MIN_NORM * _MIN_NORM, None))
    theta = jnp.clip(x_norm, -15.0, 15.0)
    e = jnp.exp(theta)
    sinh = 0.5 * (e - 1.0 / e)
    scale = sinh / x_norm
    y_sq_new = scale * scale * x_sq
    first = jnp.sqrt(jnp.clip(1.0 + y_sq_new, _EPS, None))
    col = jax.lax.broadcasted_iota(jnp.int32, u.shape, 1)
    return jnp.where(col == 0, first, u * scale)


def _gcn_kernel(adj_ref, xt_ref, hc_ref, hs_ref, cache, s_ref):
    g = pl.program_id(0)
    nb, bm, cb = _NB, _BM, _CB

    @pl.when(g < nb)
    def _():
        i = g

        @pl.when(i < cb)
        def _():
            cache[i] = adj_ref[...].astype(jnp.bfloat16)

        acc = jnp.dot(adj_ref[...], xt_ref[...], preferred_element_type=jnp.float32)
        s_ref[pl.ds(i * bm, bm), :] = acc + xt_ref[pl.ds(i * bm, bm), :]

    @pl.when(g >= nb)
    def _():
        i = g - nb + cb
        acc = jnp.dot(adj_ref[...], s_ref[...], preferred_element_type=jnp.float32)
        u = acc + s_ref[pl.ds(i * bm, bm), :] - xt_ref[pl.ds(i * bm, bm), :]
        hs_ref[...] = _expmap0_proj(u)

        j = g - nb

        @pl.when(j < cb)
        def _():
            acc2 = jnp.dot(cache[j], s_ref[...].astype(jnp.bfloat16), preferred_element_type=jnp.float32)
            u2 = acc2 + s_ref[pl.ds(j * bm, bm), :] - xt_ref[pl.ds(j * bm, bm), :]
            hc_ref[...] = _expmap0_proj(u2)


def kernel(x, adj):
    n, d = x.shape
    nb, bm, cb = _NB, _BM, _CB
    bp = 1000
    xt = pl.pallas_call(
        _logmap0_kernel,
        grid=(n // bp,),
        in_specs=[pl.BlockSpec((bp, d), lambda i: (i, 0))],
        out_specs=pl.BlockSpec((bp, d), lambda i: (i, 0)),
        out_shape=jax.ShapeDtypeStruct((n, d), jnp.float32),
    )(x)
    hc, hs = pl.pallas_call(
        _gcn_kernel,
        grid=(2 * nb - cb,),
        in_specs=[
            pl.BlockSpec((bm, n), lambda g: (jnp.where(g < nb, g, g - nb + cb), 0)),
            pl.BlockSpec((n, d), lambda g: (0, 0)),
        ],
        out_specs=[
            pl.BlockSpec(
                (bm, d),
                lambda g: (jnp.minimum(jnp.maximum(g - nb, 0), cb - 1), 0),
            ),
            pl.BlockSpec((bm, d), lambda g: (jnp.where(g < nb, 0, g - nb), 0)),
        ],
        out_shape=[
            jax.ShapeDtypeStruct((cb * bm, d), jnp.float32),
            jax.ShapeDtypeStruct((n - cb * bm, d), jnp.float32),
        ],
        scratch_shapes=[
            pltpu.VMEM((_CB, _BM, _N), jnp.bfloat16),
            pltpu.VMEM((_N, _D), jnp.float32),
        ],
        compiler_params=pltpu.CompilerParams(
            dimension_semantics=("arbitrary",),
            vmem_limit_bytes=67000000,
        ),
    )(adj, xt)
    return jnp.concatenate([hc, hs], axis=0)


# speedup vs baseline: 1.0541x; 1.0140x over previous
import jax
import jax.numpy as jnp
from jax.experimental import pallas as pl
from jax.experimental.pallas import tpu as pltpu

_MIN_NORM = 1e-15
_EPS = 1e-7

_N, _D = 10000, 128
_BM = 200
_NB = _N // _BM
_CB = 9


def _logmap0_kernel(x_ref, o_ref):
    p = x_ref[...]
    p0 = p[:, 0:1]
    y_sq = jnp.sum(p * p, axis=1, keepdims=True) - p0 * p0
    y_norm = jnp.sqrt(jnp.clip(y_sq, _MIN_NORM * _MIN_NORM, None))
    th = jnp.clip(p0, 1.0 + _EPS, None)
    ar = jnp.log(jnp.clip(th + jnp.sqrt(th * th - 1.0), _MIN_NORM, None))
    s = ar / y_norm
    col = jax.lax.broadcasted_iota(jnp.int32, p.shape, 1)
    o_ref[...] = jnp.where(col == 0, 0.0, p * s)


def _expmap0_proj(u):
    u0 = u[:, 0:1]
    x_sq = jnp.sum(u * u, axis=1, keepdims=True) - u0 * u0
    x_norm = jnp.sqrt(jnp.clip(x_sq, _MIN_NORM * _MIN_NORM, None))
    theta = jnp.clip(x_norm, -15.0, 15.0)
    e = jnp.exp(theta)
    sinh = 0.5 * (e - 1.0 / e)
    scale = sinh / x_norm
    y_sq_new = scale * scale * x_sq
    first = jnp.sqrt(jnp.clip(1.0 + y_sq_new, _EPS, None))
    col = jax.lax.broadcasted_iota(jnp.int32, u.shape, 1)
    return jnp.where(col == 0, first, u * scale)


def _gcn_kernel(adj_ref, xt_ref, h_hbm, cache, s_ref, stage, sem):
    g = pl.program_id(0)
    nb, bm, cb = _NB, _BM, _CB

    def h_copy(blk, slot):
        return pltpu.make_async_copy(
            stage.at[slot], h_hbm.at[pl.ds(blk * bm, bm), :], sem.at[slot]
        )

    @pl.when(g < nb)
    def _():
        i = g

        @pl.when(i < cb)
        def _():
            cache[i] = adj_ref[...].astype(jnp.bfloat16)

        acc = jnp.dot(adj_ref[...], xt_ref[...], preferred_element_type=jnp.float32)
        s_ref[pl.ds(i * bm, bm), :] = acc + xt_ref[pl.ds(i * bm, bm), :]

    @pl.when(g >= nb)
    def _():
        k = g - nb
        i = k + cb
        acc = jnp.dot(adj_ref[...], s_ref[...], preferred_element_type=jnp.float32)
        u = acc + s_ref[pl.ds(i * bm, bm), :] - xt_ref[pl.ds(i * bm, bm), :]
        slot = jax.lax.rem(k, 2)

        @pl.when(k >= 2)
        def _():
            h_copy(i - 2, slot).wait()

        stage[slot] = _expmap0_proj(u)
        h_copy(i, slot).start()

        j = k

        @pl.when(j < cb)
        def _():
            acc2 = jnp.dot(
                cache[j],
                s_ref[...].astype(jnp.bfloat16),
                preferred_element_type=jnp.float32,
            )
            u2 = acc2 + s_ref[pl.ds(j * bm, bm), :] - xt_ref[pl.ds(j * bm, bm), :]
            slot2 = 2 + jax.lax.rem(j, 2)

            @pl.when(j >= 2)
            def _():
                h_copy(j - 2, slot2).wait()

            stage[slot2] = _expmap0_proj(u2)
            h_copy(j, slot2).start()

    @pl.when(g == 2 * nb - cb - 1)
    def _():
        last_k = nb - cb - 1
        h_copy(last_k + cb, jax.lax.rem(last_k, 2)).wait()
        h_copy(last_k - 1 + cb, jax.lax.rem(last_k - 1, 2)).wait()
        h_copy(cb - 1, 2 + jax.lax.rem(cb - 1, 2)).wait()
        h_copy(cb - 2, 2 + jax.lax.rem(cb - 2, 2)).wait()


def kernel(x, adj):
    n, d = x.shape
    nb, bm, cb = _NB, _BM, _CB
    bp = 1000
    xt = pl.pallas_call(
        _logmap0_kernel,
        grid=(n // bp,),
        in_specs=[pl.BlockSpec((bp, d), lambda i: (i, 0))],
        out_specs=pl.BlockSpec((bp, d), lambda i: (i, 0)),
        out_shape=jax.ShapeDtypeStruct((n, d), jnp.float32),
    )(x)
    h = pl.pallas_call(
        _gcn_kernel,
        grid=(2 * nb - cb,),
        in_specs=[
            pl.BlockSpec((bm, n), lambda g: (jnp.where(g < nb, g, g - nb + cb), 0)),
            pl.BlockSpec((n, d), lambda g: (0, 0)),
        ],
        out_specs=pl.BlockSpec(memory_space=pl.ANY),
        out_shape=jax.ShapeDtypeStruct((n, d), jnp.float32),
        scratch_shapes=[
            pltpu.VMEM((_CB, _BM, _N), jnp.bfloat16),
            pltpu.VMEM((_N, _D), jnp.float32),
            pltpu.VMEM((4, _BM, _D), jnp.float32),
            pltpu.SemaphoreType.DMA((4,)),
        ],
        compiler_params=pltpu.CompilerParams(
            dimension_semantics=("arbitrary",),
            vmem_limit_bytes=67000000,
        ),
    )(adj, xt)
    return h


# speedup vs baseline: 1.0609x; 1.0065x over previous
import jax
import jax.numpy as jnp
from jax.experimental import pallas as pl
from jax.experimental.pallas import tpu as pltpu

_MIN_NORM = 1e-15
_EPS = 1e-7

_N, _D = 10000, 128
_BM = 200
_NB = _N // _BM
_CB = 9


def _logmap0_kernel(x_ref, o_ref):
    p = x_ref[...]
    p0 = p[:, 0:1]
    y_sq = jnp.sum(p * p, axis=1, keepdims=True) - p0 * p0
    y_norm = jnp.sqrt(jnp.clip(y_sq, _MIN_NORM * _MIN_NORM, None))
    th = jnp.clip(p0, 1.0 + _EPS, None)
    ar = jnp.log(jnp.clip(th + jnp.sqrt(th * th - 1.0), _MIN_NORM, None))
    s = ar / y_norm
    col = jax.lax.broadcasted_iota(jnp.int32, p.shape, 1)
    o_ref[...] = jnp.where(col == 0, 0.0, p * s)


def _expmap0_proj(u):
    u0 = u[:, 0:1]
    x_sq = jnp.sum(u * u, axis=1, keepdims=True) - u0 * u0
    x_norm = jnp.sqrt(jnp.clip(x_sq, _MIN_NORM * _MIN_NORM, None))
    theta = jnp.clip(x_norm, -15.0, 15.0)
    e = jnp.exp(theta)
    sinh = 0.5 * (e - 1.0 / e)
    scale = sinh / x_norm
    y_sq_new = scale * scale * x_sq
    first = jnp.sqrt(jnp.clip(1.0 + y_sq_new, _EPS, None))
    col = jax.lax.broadcasted_iota(jnp.int32, u.shape, 1)
    return jnp.where(col == 0, first, u * scale)


def _gcn_kernel(adj_ref, xt_ref, h_hbm, cache, s_ref, stage, sem):
    g = pl.program_id(0)
    nb, bm, cb = _NB, _BM, _CB

    def h_copy(blk, slot):
        return pltpu.make_async_copy(
            stage.at[slot], h_hbm.at[pl.ds(blk * bm, bm), :], sem.at[slot]
        )

    @pl.when(g < nb)
    def _():
        i = g

        @pl.when(i < cb)
        def _():
            cache[i] = adj_ref[...].astype(jnp.bfloat16)

        acc = jnp.dot(adj_ref[...], xt_ref[...], preferred_element_type=jnp.float32)
        s_ref[pl.ds(i * bm, bm), :] = acc + xt_ref[pl.ds(i * bm, bm), :]

    def blk_of(k):
        return jnp.where(k == 0, nb - 1, k - 1 + cb)

    @pl.when(g >= nb)
    def _():
        k = g - nb
        i = blk_of(k)
        acc = jnp.dot(adj_ref[...], s_ref[...], preferred_element_type=jnp.float32)
        u = acc + s_ref[pl.ds(i * bm, bm), :] - xt_ref[pl.ds(i * bm, bm), :]
        slot = jax.lax.rem(k, 2)

        @pl.when(k >= 2)
        def _():
            h_copy(blk_of(k - 2), slot).wait()

        stage[slot] = _expmap0_proj(u)
        h_copy(i, slot).start()

        j = k

        @pl.when(j < cb)
        def _():
            acc2 = jnp.dot(
                cache[j],
                s_ref[...].astype(jnp.bfloat16),
                preferred_element_type=jnp.float32,
            )
            u2 = acc2 + s_ref[pl.ds(j * bm, bm), :] - xt_ref[pl.ds(j * bm, bm), :]
            slot2 = 2 + jax.lax.rem(j, 2)

            @pl.when(j >= 2)
            def _():
                h_copy(j - 2, slot2).wait()

            stage[slot2] = _expmap0_proj(u2)
            h_copy(j, slot2).start()

    @pl.when(g == 2 * nb - cb - 1)
    def _():
        last_k = nb - cb - 1
        h_copy(blk_of(last_k), jax.lax.rem(last_k, 2)).wait()
        h_copy(blk_of(last_k - 1), jax.lax.rem(last_k - 1, 2)).wait()
        h_copy(cb - 1, 2 + jax.lax.rem(cb - 1, 2)).wait()
        h_copy(cb - 2, 2 + jax.lax.rem(cb - 2, 2)).wait()


def kernel(x, adj):
    n, d = x.shape
    nb, bm, cb = _NB, _BM, _CB
    bp = 1000
    xt = pl.pallas_call(
        _logmap0_kernel,
        grid=(n // bp,),
        in_specs=[pl.BlockSpec((bp, d), lambda i: (i, 0))],
        out_specs=pl.BlockSpec((bp, d), lambda i: (i, 0)),
        out_shape=jax.ShapeDtypeStruct((n, d), jnp.float32),
    )(x)
    h = pl.pallas_call(
        _gcn_kernel,
        grid=(2 * nb - cb,),
        in_specs=[
            pl.BlockSpec(
                (bm, n),
                lambda g: (
                    jnp.where(
                        g < nb, g, jnp.where(g == nb, nb - 1, g - nb - 1 + cb)
                    ),
                    0,
                ),
            ),
            pl.BlockSpec((n, d), lambda g: (0, 0)),
        ],
        out_specs=pl.BlockSpec(memory_space=pl.ANY),
        out_shape=jax.ShapeDtypeStruct((n, d), jnp.float32),
        scratch_shapes=[
            pltpu.VMEM((_CB, _BM, _N), jnp.bfloat16),
            pltpu.VMEM((_N, _D), jnp.float32),
            pltpu.VMEM((4, _BM, _D), jnp.float32),
            pltpu.SemaphoreType.DMA((4,)),
        ],
        compiler_params=pltpu.CompilerParams(
            dimension_semantics=("arbitrary",),
            vmem_limit_bytes=67000000,
        ),
    )(adj, xt)
    return h


# speedup vs baseline: 1.0714x; 1.0098x over previous
import jax
import jax.numpy as jnp
from jax.experimental import pallas as pl
from jax.experimental.pallas import tpu as pltpu

_MIN_NORM = 1e-15
_EPS = 1e-7

_N, _D = 10000, 128
_BM = 200
_NB = _N // _BM
_CB = 8


def _logmap0(p):
    p0 = p[:, 0:1]
    y_sq = jnp.sum(p * p, axis=1, keepdims=True) - p0 * p0
    y_norm = jnp.sqrt(jnp.clip(y_sq, _MIN_NORM * _MIN_NORM, None))
    th = jnp.clip(p0, 1.0 + _EPS, None)
    ar = jnp.log(jnp.clip(th + jnp.sqrt(th * th - 1.0), _MIN_NORM, None))
    s = ar / y_norm
    col = jax.lax.broadcasted_iota(jnp.int32, p.shape, 1)
    return jnp.where(col == 0, 0.0, p * s)


def _expmap0_proj(u):
    u0 = u[:, 0:1]
    x_sq = jnp.sum(u * u, axis=1, keepdims=True) - u0 * u0
    x_norm = jnp.sqrt(jnp.clip(x_sq, _MIN_NORM * _MIN_NORM, None))
    theta = jnp.clip(x_norm, -15.0, 15.0)
    e = jnp.exp(theta)
    sinh = 0.5 * (e - 1.0 / e)
    scale = sinh / x_norm
    y_sq_new = scale * scale * x_sq
    first = jnp.sqrt(jnp.clip(1.0 + y_sq_new, _EPS, None))
    col = jax.lax.broadcasted_iota(jnp.int32, u.shape, 1)
    return jnp.where(col == 0, first, u * scale)


def _gcn_kernel(adj_ref, x_ref, h_hbm, cache, xt_ref, s_ref, stage, sem):
    g = pl.program_id(0)
    nb, bm, cb = _NB, _BM, _CB

    def h_copy(blk, slot):
        return pltpu.make_async_copy(
            stage.at[slot], h_hbm.at[pl.ds(blk * bm, bm), :], sem.at[slot]
        )

    @pl.when(g == 0)
    def _():
        def body(t, _):
            r = pl.ds(t * 1000, 1000)
            xt_ref[r, :] = _logmap0(x_ref[r, :])
            return 0

        jax.lax.fori_loop(0, _N // 1000, body, 0, unroll=False)

    @pl.when(jnp.logical_and(g >= 1, g <= nb))
    def _():
        i = g - 1

        @pl.when(i < cb)
        def _():
            cache[i] = adj_ref[...].astype(jnp.bfloat16)

        acc = jnp.dot(adj_ref[...], xt_ref[...], preferred_element_type=jnp.float32)
        s_ref[pl.ds(i * bm, bm), :] = acc + xt_ref[pl.ds(i * bm, bm), :]

    def blk_of(k):
        return jnp.where(k == 0, nb - 1, k - 1 + cb)

    @pl.when(g > nb)
    def _():
        k = g - nb - 1
        i = blk_of(k)
        acc = jnp.dot(adj_ref[...], s_ref[...], preferred_element_type=jnp.float32)
        u = acc + s_ref[pl.ds(i * bm, bm), :] - xt_ref[pl.ds(i * bm, bm), :]
        slot = jax.lax.rem(k, 2)

        @pl.when(k >= 2)
        def _():
            h_copy(blk_of(k - 2), slot).wait()

        stage[slot] = _expmap0_proj(u)
        h_copy(i, slot).start()

        j = k

        @pl.when(j < cb)
        def _():
            acc2 = jnp.dot(
                cache[j],
                s_ref[...].astype(jnp.bfloat16),
                preferred_element_type=jnp.float32,
            )
            u2 = acc2 + s_ref[pl.ds(j * bm, bm), :] - xt_ref[pl.ds(j * bm, bm), :]
            slot2 = 2 + jax.lax.rem(j, 2)

            @pl.when(j >= 2)
            def _():
                h_copy(j - 2, slot2).wait()

            stage[slot2] = _expmap0_proj(u2)
            h_copy(j, slot2).start()

    @pl.when(g == 2 * nb - cb)
    def _():
        last_k = nb - cb - 1
        h_copy(blk_of(last_k), jax.lax.rem(last_k, 2)).wait()
        h_copy(blk_of(last_k - 1), jax.lax.rem(last_k - 1, 2)).wait()
        h_copy(cb - 1, 2 + jax.lax.rem(cb - 1, 2)).wait()
        h_copy(cb - 2, 2 + jax.lax.rem(cb - 2, 2)).wait()


def kernel(x, adj):
    n, d = x.shape
    nb, bm, cb = _NB, _BM, _CB

    def adj_index(g):
        k = g - nb - 1
        return (
            jnp.where(
                g <= nb,
                jnp.maximum(g - 1, 0),
                jnp.where(k == 0, nb - 1, k - 1 + cb),
            ),
            0,
        )

    h = pl.pallas_call(
        _gcn_kernel,
        grid=(2 * nb - cb + 1,),
        in_specs=[
            pl.BlockSpec((bm, n), adj_index),
            pl.BlockSpec((n, d), lambda g: (0, 0)),
        ],
        out_specs=pl.BlockSpec(memory_space=pl.ANY),
        out_shape=jax.ShapeDtypeStruct((n, d), jnp.float32),
        scratch_shapes=[
            pltpu.VMEM((_CB, _BM, _N), jnp.bfloat16),
            pltpu.VMEM((_N, _D), jnp.float32),
            pltpu.VMEM((_N, _D), jnp.float32),
            pltpu.VMEM((4, _BM, _D), jnp.float32),
            pltpu.SemaphoreType.DMA((4,)),
        ],
        compiler_params=pltpu.CompilerParams(
            dimension_semantics=("arbitrary",),
            vmem_limit_bytes=67000000,
        ),
    )(adj, x)
    return h


# speedup vs baseline: 1.0879x; 1.0155x over previous
import jax
import jax.numpy as jnp
from jax.experimental import pallas as pl
from jax.experimental.pallas import tpu as pltpu

_MIN_NORM = 1e-15
_EPS = 1e-7

_N, _D = 10000, 128
_BM = 200
_NB = _N // _BM
_CB = 9


def _logmap0(p):
    p0 = p[:, 0:1]
    y_sq = jnp.sum(p * p, axis=1, keepdims=True) - p0 * p0
    y_norm = jnp.sqrt(jnp.clip(y_sq, _MIN_NORM * _MIN_NORM, None))
    th = jnp.clip(p0, 1.0 + _EPS, None)
    ar = jnp.log(jnp.clip(th + jnp.sqrt(th * th - 1.0), _MIN_NORM, None))
    s = ar / y_norm
    col = jax.lax.broadcasted_iota(jnp.int32, p.shape, 1)
    return jnp.where(col == 0, 0.0, p * s)


def _expmap0_proj(u):
    u0 = u[:, 0:1]
    x_sq = jnp.sum(u * u, axis=1, keepdims=True) - u0 * u0
    x_norm = jnp.sqrt(jnp.clip(x_sq, _MIN_NORM * _MIN_NORM, None))
    theta = jnp.clip(x_norm, -15.0, 15.0)
    e = jnp.exp(theta)
    sinh = 0.5 * (e - 1.0 / e)
    scale = sinh / x_norm
    y_sq_new = scale * scale * x_sq
    first = jnp.sqrt(jnp.clip(1.0 + y_sq_new, _EPS, None))
    col = jax.lax.broadcasted_iota(jnp.int32, u.shape, 1)
    return jnp.where(col == 0, first, u * scale)


def _gcn_kernel(adj_ref, x_hbm, h_hbm, cache, xt_ref, s_ref, stage, sem):
    g = pl.program_id(0)
    nb, bm, cb = _NB, _BM, _CB

    def h_copy(blk, slot):
        return pltpu.make_async_copy(
            stage.at[slot], h_hbm.at[pl.ds(blk * bm, bm), :], sem.at[slot]
        )

    @pl.when(g == 0)
    def _():
        cp = pltpu.make_async_copy(x_hbm, xt_ref, sem.at[0])
        cp.start()
        cp.wait()

        def body(t, _):
            r = pl.ds(t * 1000, 1000)
            xt_ref[r, :] = _logmap0(xt_ref[r, :])
            return 0

        jax.lax.fori_loop(0, _N // 1000, body, 0, unroll=False)

    @pl.when(jnp.logical_and(g >= 1, g <= nb))
    def _():
        i = g - 1

        @pl.when(i < cb)
        def _():
            cache[i] = adj_ref[...].astype(jnp.bfloat16)

        acc = jnp.dot(adj_ref[...], xt_ref[...], preferred_element_type=jnp.float32)
        s_ref[pl.ds(i * bm, bm), :] = acc + xt_ref[pl.ds(i * bm, bm), :]

    def blk_of(k):
        return jnp.where(k == 0, nb - 1, k - 1 + cb)

    @pl.when(g > nb)
    def _():
        k = g - nb - 1
        i = blk_of(k)
        acc = jnp.dot(adj_ref[...], s_ref[...], preferred_element_type=jnp.float32)
        u = acc + s_ref[pl.ds(i * bm, bm), :] - xt_ref[pl.ds(i * bm, bm), :]
        slot = jax.lax.rem(k, 2)

        @pl.when(k >= 2)
        def _():
            h_copy(blk_of(k - 2), slot).wait()

        stage[slot] = _expmap0_proj(u)
        h_copy(i, slot).start()

        j = k

        @pl.when(j < cb)
        def _():
            acc2 = jnp.dot(
                cache[j],
                s_ref[...].astype(jnp.bfloat16),
                preferred_element_type=jnp.float32,
            )
            u2 = acc2 + s_ref[pl.ds(j * bm, bm), :] - xt_ref[pl.ds(j * bm, bm), :]
            slot2 = 2 + jax.lax.rem(j, 2)

            @pl.when(j >= 2)
            def _():
                h_copy(j - 2, slot2).wait()

            stage[slot2] = _expmap0_proj(u2)
            h_copy(j, slot2).start()

    @pl.when(g == 2 * nb - cb)
    def _():
        last_k = nb - cb - 1
        h_copy(blk_of(last_k), jax.lax.rem(last_k, 2)).wait()
        h_copy(blk_of(last_k - 1), jax.lax.rem(last_k - 1, 2)).wait()
        h_copy(cb - 1, 2 + jax.lax.rem(cb - 1, 2)).wait()
        h_copy(cb - 2, 2 + jax.lax.rem(cb - 2, 2)).wait()


def kernel(x, adj):
    n, d = x.shape
    nb, bm, cb = _NB, _BM, _CB

    def adj_index(g):
        k = g - nb - 1
        return (
            jnp.where(
                g <= nb,
                jnp.maximum(g - 1, 0),
                jnp.where(k == 0, nb - 1, k - 1 + cb),
            ),
            0,
        )

    h = pl.pallas_call(
        _gcn_kernel,
        grid=(2 * nb - cb + 1,),
        in_specs=[
            pl.BlockSpec((bm, n), adj_index),
            pl.BlockSpec(memory_space=pl.ANY),
        ],
        out_specs=pl.BlockSpec(memory_space=pl.ANY),
        out_shape=jax.ShapeDtypeStruct((n, d), jnp.float32),
        scratch_shapes=[
            pltpu.VMEM((_CB, _BM, _N), jnp.bfloat16),
            pltpu.VMEM((_N, _D), jnp.float32),
            pltpu.VMEM((_N, _D), jnp.float32),
            pltpu.VMEM((4, _BM, _D), jnp.float32),
            pltpu.SemaphoreType.DMA((4,)),
        ],
        compiler_params=pltpu.CompilerParams(
            dimension_semantics=("arbitrary",),
            vmem_limit_bytes=67000000,
        ),
    )(adj, x)
    return h
